# baseline (speedup 1.0000x reference)
import jax
import jax.numpy as jnp
from jax.experimental import pallas as pl
from jax.experimental.pallas import tpu as pltpu

DIM = 512
HEADS = 8
DH = DIM // HEADS
STEPS = 12
F32 = jnp.float32
BF16 = jnp.bfloat16
BB = 8


def _dot(a, b):
    return jax.lax.dot_general(a, b, (((1,), (0,)), ((), ())),
                               preferred_element_type=F32)


def _dot_tb(a, b):
    return jax.lax.dot_general(a, b, (((1,), (1,)), ((), ())),
                               preferred_element_type=F32)


def _prep_kernel(question, pos_w, pos_b, mem_w, know_w, rcw, rcpm_w, rcpm_b,
                 wcw, wcpm_w, wcpm_b, wkh_m, bk_m, wkh_k, bk_k, wkh_r, bk_r,
                 wkh_w, bk_w, pa_out, khm_out, khk_out, khr_out, khw_out):
    q = question[...]
    for i in range(STEPS):
        pa_out[i] = _dot_tb(q, pos_w[i]) + pos_b[i]
    cpm = _dot_tb(rcw[...], rcpm_w[...]) + rcpm_b[...]
    wcpm = _dot_tb(wcw[...], wcpm_w[...]) + wcpm_b[...]
    for h in range(HEADS):
        khm_out[h] = _dot_tb(wkh_m[h], mem_w[...]) + bk_m[h]
        khk_out[h] = _dot_tb(wkh_k[h], know_w[...]) + bk_k[h]
        khr_out[h] = _dot_tb(wkh_r[h], cpm) + bk_r[h]
        khw_out[h] = _dot_tb(wkh_w[h], wcpm) + bk_w[h]


def _mha_mask(ctrl, wqT, bq, kh):
    q = (_dot(ctrl, wqT[...]) + bq[...]) * (DH ** -0.5)
    acc = None
    for h in range(HEADS):
        s = _dot(q[:, h * DH:(h + 1) * DH], kh[h])
        p = jax.nn.softmax(s, axis=-1)
        acc = p if acc is None else acc + p
    return acc * (1.0 / HEADS)


def _mac_kernel(ctxT_ref, pa_ref, know_hbm, c0, m0, cqw1T, cqw2T,
                cqb, cattnw,
                memwT, memb, wqTm, bqm, khm, wqTk, bqk, khk, wqTr, bqr, khr,
                wqTw, bqw, khw, knoww, rcw, rattnw, wcw1T, wcw2T, wcb,
                out_ref, know_ref, sem):
    bb = ctxT_ref.shape[0]
    g = pl.program_id(0)
    cp = pltpu.make_async_copy(know_hbm.at[pl.ds(g * bb, bb)], know_ref, sem)
    cp.start()

    def _ctrl_update(control, i):
        pa = pa_ref[i, pl.ds(g * bb, bb), :]
        cq = _dot(control, cqw1T[...]) + _dot(pa, cqw2T[...]) + cqb[...]
        cqw = cq * cattnw[...]
        aw = jnp.concatenate(
            [_dot(cqw[b:b + 1], ctxT_ref[b]) for b in range(bb)], axis=0)
        pA = jax.nn.softmax(aw, axis=-1)
        return jnp.concatenate(
            [_dot_tb(pA[b:b + 1], ctxT_ref[b]) for b in range(bb)], axis=0)

    def _masks(control):
        return (_mha_mask(control, wqTm, bqm, khm),
                _mha_mask(control, wqTk, bqk, khk),
                _mha_mask(control, wqTr, bqr, khr),
                _mha_mask(control, wqTw, bqw, khw))

    def _memory_chain(control, memory, m_mem, m_kno, m_rca, m_wca):
        mem_proj = (_dot(memory, memwT[...]) + memb[...]) * m_mem
        wv = m_rca * control * rattnw[...]
        t = _dot(wv, rcw[...])
        z = _dot(mem_proj * m_kno * t[:, :DIM], knoww[...]) + t[:, DIM:]
        zb = z.astype(BF16)
        logits = jnp.concatenate(
            [_dot(zb[b:b + 1], know_ref[b]) for b in range(bb)], axis=0)
        pR = jax.nn.softmax(logits, axis=-1).astype(BF16)
        read = jnp.concatenate(
            [_dot_tb(pR[b:b + 1], know_ref[b]) for b in range(bb)], axis=0)
        wc = _dot(read, wcw1T[...]) + _dot(memory, wcw2T[...]) + wcb[...]
        return wc * m_wca

    control = _ctrl_update(jnp.broadcast_to(c0[...], (bb, DIM)), 0)
    masks = _masks(control)
    cp.wait()

    def _step(i, carry):
        control, memory, m_mem, m_kno, m_rca, m_wca = carry
        memory = _memory_chain(control, memory, m_mem, m_kno, m_rca, m_wca)
        ncontrol = _ctrl_update(control, i + 1)
        return (ncontrol, memory) + _masks(ncontrol)

    carry = jax.lax.fori_loop(
        0, STEPS - 1, _step,
        (control, jnp.broadcast_to(m0[...], (bb, DIM))) + masks)
    out_ref[...] = _memory_chain(carry[0], carry[1], *carry[2:])


def kernel(context, question, knowledge, params):
    b = question.shape[0]
    c, r, w = params['ctrl'], params['read'], params['write']
    mm, km, rm, wm = (r['mem_mask'], r['know_mask'], r['concat_mask'],
                      w['concat_mask'])

    def row(v):
        return v.reshape(1, -1)

    prep = pl.pallas_call(
        _prep_kernel,
        out_shape=(
            jax.ShapeDtypeStruct((STEPS, b, DIM), F32),
            jax.ShapeDtypeStruct((HEADS, DH, DIM), F32),
            jax.ShapeDtypeStruct((HEADS, DH, DIM), F32),
            jax.ShapeDtypeStruct((HEADS, DH, DIM), F32),
            jax.ShapeDtypeStruct((HEADS, DH, DIM), F32),
        ),
        compiler_params=pltpu.CompilerParams(
            vmem_limit_bytes=60 * 1024 * 1024),
        name="mac_prep",
    )
    pa_all, khm, khk, khr, khw = prep(
        question, c['pos_w'], c['pos_b'].reshape(STEPS, 1, DIM),
        r['mem_w'], r['know_w'], r['concat_w'], r['cpm_w'], row(r['cpm_b']),
        w['concat_w'], w['cpm_w'], row(w['cpm_b']),
        mm['wk'].reshape(HEADS, DH, DIM), mm['bk'].reshape(HEADS, DH, 1),
        km['wk'].reshape(HEADS, DH, DIM), km['bk'].reshape(HEADS, DH, 1),
        rm['wk'].reshape(HEADS, DH, DIM), rm['bk'].reshape(HEADS, DH, 1),
        wm['wk'].reshape(HEADS, DH, DIM), wm['bk'].reshape(HEADS, DH, 1),
    )

    nb = b // BB
    S = context.shape[1]
    K = knowledge.shape[2]
    vmem = pl.BlockSpec(memory_space=pltpu.VMEM)
    out = pl.pallas_call(
        _mac_kernel,
        grid=(nb,),
        in_specs=[
            pl.BlockSpec((BB, DIM, S), lambda i: (i, 0, 0)),
            vmem,
            pl.BlockSpec(memory_space=pl.ANY),
        ] + [vmem] * 26,
        out_specs=pl.BlockSpec((BB, DIM), lambda i: (i, 0)),
        out_shape=jax.ShapeDtypeStruct((b, DIM), F32),
        scratch_shapes=[pltpu.VMEM((BB, DIM, K), BF16),
                        pltpu.SemaphoreType.DMA],
        compiler_params=pltpu.CompilerParams(
            dimension_semantics=("parallel",),
            vmem_limit_bytes=60 * 1024 * 1024),
        name="mac_steps",
    )
    return out(
        context.transpose(0, 2, 1), pa_all, knowledge.astype(BF16),
        params['control_0'], params['mem_0'],
        c['cq_w'][:, :DIM].T, c['cq_w'][:, DIM:].T, row(c['cq_b']),
        c['attn_w'],
        r['mem_w'].T, row(r['mem_b']),
        mm['wq'].T, row(mm['bq']), khm,
        km['wq'].T, row(km['bq']), khk,
        rm['wq'].T, row(rm['bq']), khr,
        wm['wq'].T, row(wm['bq']), khw,
        r['know_w'], r['concat_w'], r['attn_w'],
        w['concat_w'][:, :DIM].T, w['concat_w'][:, DIM:].T, row(w['concat_b']))


# speedup vs baseline: 1.2568x; 1.2568x over previous
import jax
import jax.numpy as jnp
from jax.experimental import pallas as pl
from jax.experimental.pallas import tpu as pltpu

DIM = 512
HEADS = 8
DH = DIM // HEADS
STEPS = 12
F32 = jnp.float32
BF16 = jnp.bfloat16
BB = 16


def _dot(a, b):
    return jax.lax.dot_general(a, b, (((1,), (0,)), ((), ())),
                               preferred_element_type=F32)


def _dot_tb(a, b):
    return jax.lax.dot_general(a, b, (((1,), (1,)), ((), ())),
                               preferred_element_type=F32)


def _prep_kernel(question, pos_w, pos_b, mem_w, know_w, rcw, rcpm_w, rcpm_b,
                 wcw, wcpm_w, wcpm_b, wkh_m, bk_m, wkh_k, bk_k, wkh_r, bk_r,
                 wkh_w, bk_w, pa_out, khm_out, khk_out, khr_out, khw_out):
    q = question[...]
    for i in range(STEPS):
        pa_out[i] = _dot_tb(q, pos_w[i]) + pos_b[i]
    cpm = _dot_tb(rcw[...], rcpm_w[...]) + rcpm_b[...]
    wcpm = _dot_tb(wcw[...], wcpm_w[...]) + wcpm_b[...]
    for h in range(HEADS):
        khm_out[h] = _dot_tb(wkh_m[h], mem_w[...]) + bk_m[h]
        khk_out[h] = _dot_tb(wkh_k[h], know_w[...]) + bk_k[h]
        khr_out[h] = _dot_tb(wkh_r[h], cpm) + bk_r[h]
        khw_out[h] = _dot_tb(wkh_w[h], wcpm) + bk_w[h]


def _mha_mask(ctrl, wqT, bq, kh):
    q = (_dot(ctrl, wqT[...]) + bq[...]) * (DH ** -0.5)
    acc = None
    for h in range(HEADS):
        s = _dot(q[:, h * DH:(h + 1) * DH], kh[h])
        p = jax.nn.softmax(s, axis=-1)
        acc = p if acc is None else acc + p
    return acc * (1.0 / HEADS)


def _mac_kernel(ctxT_ref, pa_ref, know_hbm, c0, m0, cqw1T, cqw2T,
                cqb, cattnw,
                memwT, memb, wqTm, bqm, khm, wqTk, bqk, khk, wqTr, bqr, khr,
                wqTw, bqw, khw, knoww, rcw, rattnw, wcw1T, wcw2T, wcb,
                out_ref, know_ref, sem):
    bb = ctxT_ref.shape[0]
    g = pl.program_id(0)
    cp = pltpu.make_async_copy(know_hbm.at[pl.ds(g * bb, bb)], know_ref, sem)
    cp.start()

    control = jnp.broadcast_to(c0[...], (bb, DIM))
    memory = jnp.broadcast_to(m0[...], (bb, DIM))
    cp.wait()

    def _step(i, carry):
        control, memory = carry
        pa = pa_ref[i, pl.ds(g * bb, bb), :]
        cq = _dot(control, cqw1T[...]) + _dot(pa, cqw2T[...]) + cqb[...]
        cqw = cq * cattnw[...]
        aw = jnp.concatenate(
            [_dot(cqw[b:b + 1], ctxT_ref[b]) for b in range(bb)], axis=0)
        pA = jax.nn.softmax(aw, axis=-1)
        control = jnp.concatenate(
            [_dot_tb(pA[b:b + 1], ctxT_ref[b]) for b in range(bb)], axis=0)
        m_mem = _mha_mask(control, wqTm, bqm, khm)
        m_kno = _mha_mask(control, wqTk, bqk, khk)
        m_rca = _mha_mask(control, wqTr, bqr, khr)
        m_wca = _mha_mask(control, wqTw, bqw, khw)
        mem_proj = (_dot(memory, memwT[...]) + memb[...]) * m_mem
        wv = m_rca * control * rattnw[...]
        t = _dot(wv, rcw[...])
        z = _dot(mem_proj * m_kno * t[:, :DIM], knoww[...]) + t[:, DIM:]
        zb = z.astype(BF16)
        logits = jnp.concatenate(
            [_dot(zb[b:b + 1], know_ref[b]) for b in range(bb)], axis=0)
        pR = jax.nn.softmax(logits, axis=-1).astype(BF16)
        read = jnp.concatenate(
            [_dot_tb(pR[b:b + 1], know_ref[b]) for b in range(bb)], axis=0)
        wc = _dot(read, wcw1T[...]) + _dot(memory, wcw2T[...]) + wcb[...]
        memory = wc * m_wca
        return control, memory

    control, memory = jax.lax.fori_loop(
        0, STEPS, _step, (control, memory))
    out_ref[...] = memory


def kernel(context, question, knowledge, params):
    b = question.shape[0]
    c, r, w = params['ctrl'], params['read'], params['write']
    mm, km, rm, wm = (r['mem_mask'], r['know_mask'], r['concat_mask'],
                      w['concat_mask'])

    def row(v):
        return v.reshape(1, -1)

    prep = pl.pallas_call(
        _prep_kernel,
        out_shape=(
            jax.ShapeDtypeStruct((STEPS, b, DIM), F32),
            jax.ShapeDtypeStruct((HEADS, DH, DIM), F32),
            jax.ShapeDtypeStruct((HEADS, DH, DIM), F32),
            jax.ShapeDtypeStruct((HEADS, DH, DIM), F32),
            jax.ShapeDtypeStruct((HEADS, DH, DIM), F32),
        ),
        compiler_params=pltpu.CompilerParams(
            vmem_limit_bytes=60 * 1024 * 1024),
        name="mac_prep",
    )
    pa_all, khm, khk, khr, khw = prep(
        question, c['pos_w'], c['pos_b'].reshape(STEPS, 1, DIM),
        r['mem_w'], r['know_w'], r['concat_w'], r['cpm_w'], row(r['cpm_b']),
        w['concat_w'], w['cpm_w'], row(w['cpm_b']),
        mm['wk'].reshape(HEADS, DH, DIM), mm['bk'].reshape(HEADS, DH, 1),
        km['wk'].reshape(HEADS, DH, DIM), km['bk'].reshape(HEADS, DH, 1),
        rm['wk'].reshape(HEADS, DH, DIM), rm['bk'].reshape(HEADS, DH, 1),
        wm['wk'].reshape(HEADS, DH, DIM), wm['bk'].reshape(HEADS, DH, 1),
    )

    nb = b // BB
    S = context.shape[1]
    K = knowledge.shape[2]
    vmem = pl.BlockSpec(memory_space=pltpu.VMEM)
    out = pl.pallas_call(
        _mac_kernel,
        grid=(nb,),
        in_specs=[
            pl.BlockSpec((BB, DIM, S), lambda i: (i, 0, 0)),
            vmem,
            pl.BlockSpec(memory_space=pl.ANY),
        ] + [vmem] * 26,
        out_specs=pl.BlockSpec((BB, DIM), lambda i: (i, 0)),
        out_shape=jax.ShapeDtypeStruct((b, DIM), F32),
        scratch_shapes=[pltpu.VMEM((BB, DIM, K), BF16),
                        pltpu.SemaphoreType.DMA],
        compiler_params=pltpu.CompilerParams(
            dimension_semantics=("parallel",),
            vmem_limit_bytes=60 * 1024 * 1024),
        name="mac_steps",
    )
    return out(
        context.transpose(0, 2, 1), pa_all, knowledge.astype(BF16),
        params['control_0'], params['mem_0'],
        c['cq_w'][:, :DIM].T, c['cq_w'][:, DIM:].T, row(c['cq_b']),
        c['attn_w'],
        r['mem_w'].T, row(r['mem_b']),
        mm['wq'].T, row(mm['bq']), khm,
        km['wq'].T, row(km['bq']), khk,
        rm['wq'].T, row(rm['bq']), khr,
        wm['wq'].T, row(wm['bq']), khw,
        r['know_w'], r['concat_w'], r['attn_w'],
        w['concat_w'][:, :DIM].T, w['concat_w'][:, DIM:].T, row(w['concat_b']))


# speedup vs baseline: 1.2765x; 1.0156x over previous
import jax
import jax.numpy as jnp
from jax.experimental import pallas as pl
from jax.experimental.pallas import tpu as pltpu

DIM = 512
HEADS = 8
DH = DIM // HEADS
STEPS = 12
F32 = jnp.float32
BF16 = jnp.bfloat16
BB = 16


def _dot(a, b):
    return jax.lax.dot_general(a, b, (((1,), (0,)), ((), ())),
                               preferred_element_type=F32)


def _dot_tb(a, b):
    return jax.lax.dot_general(a, b, (((1,), (1,)), ((), ())),
                               preferred_element_type=F32)


def _bdot(a, b, cdim):
    return jax.lax.dot_general(a, b, (((1,), (cdim,)), ((0,), (0,))),
                               preferred_element_type=F32)


def _prep_kernel(question, pos_w, pos_b, mem_w, know_w, rcw, rcpm_w, rcpm_b,
                 wcw, wcpm_w, wcpm_b, wkh_m, bk_m, wkh_k, bk_k, wkh_r, bk_r,
                 wkh_w, bk_w, pa_out, khm_out, khk_out, khr_out, khw_out):
    q = question[...]
    for i in range(STEPS):
        pa_out[i] = _dot_tb(q, pos_w[i]) + pos_b[i]
    cpm = _dot_tb(rcw[...], rcpm_w[...]) + rcpm_b[...]
    wcpm = _dot_tb(wcw[...], wcpm_w[...]) + wcpm_b[...]
    for h in range(HEADS):
        khm_out[h] = _dot_tb(wkh_m[h], mem_w[...]) + bk_m[h]
        khk_out[h] = _dot_tb(wkh_k[h], know_w[...]) + bk_k[h]
        khr_out[h] = _dot_tb(wkh_r[h], cpm) + bk_r[h]
        khw_out[h] = _dot_tb(wkh_w[h], wcpm) + bk_w[h]


def _mha_mask(ctrl, wqT, bq, kh):
    q = (_dot(ctrl, wqT[...]) + bq[...]) * (DH ** -0.5)
    acc = None
    for h in range(HEADS):
        s = _dot(q[:, h * DH:(h + 1) * DH], kh[h])
        p = jax.nn.softmax(s, axis=-1)
        acc = p if acc is None else acc + p
    return acc * (1.0 / HEADS)


def _mac_kernel(ctxT_ref, pa_ref, know_hbm, c0, m0, cqw1T, cqw2T,
                cqb, cattnw,
                memwT, memb, wqTm, bqm, khm, wqTk, bqk, khk, wqTr, bqr, khr,
                wqTw, bqw, khw, knoww, rcw, rattnw, wcw1T, wcw2T, wcb,
                out_ref, know_ref, sem):
    bb = ctxT_ref.shape[0]
    g = pl.program_id(0)
    cp = pltpu.make_async_copy(know_hbm.at[pl.ds(g * bb, bb)], know_ref, sem)
    cp.start()

    control = jnp.broadcast_to(c0[...], (bb, DIM))
    memory = jnp.broadcast_to(m0[...], (bb, DIM))
    cp.wait()

    def _step(i, carry):
        control, memory = carry
        pa = pa_ref[i, pl.ds(g * bb, bb), :]
        cq = _dot(control, cqw1T[...]) + _dot(pa, cqw2T[...]) + cqb[...]
        cqw = cq * cattnw[...]
        ctxT = ctxT_ref[...]
        aw = _bdot(cqw, ctxT, 1)
        pA = jax.nn.softmax(aw, axis=-1)
        control = _bdot(pA, ctxT, 2)
        m_mem = _mha_mask(control, wqTm, bqm, khm)
        m_kno = _mha_mask(control, wqTk, bqk, khk)
        m_rca = _mha_mask(control, wqTr, bqr, khr)
        m_wca = _mha_mask(control, wqTw, bqw, khw)
        mem_proj = (_dot(memory, memwT[...]) + memb[...]) * m_mem
        wv = m_rca * control * rattnw[...]
        t = _dot(wv, rcw[...])
        z = _dot(mem_proj * m_kno * t[:, :DIM], knoww[...]) + t[:, DIM:]
        zb = z.astype(BF16)
        know = know_ref[...]
        logits = _bdot(zb, know, 1)
        pR = jax.nn.softmax(logits, axis=-1).astype(BF16)
        read = _bdot(pR, know, 2)
        wc = _dot(read, wcw1T[...]) + _dot(memory, wcw2T[...]) + wcb[...]
        memory = wc * m_wca
        return control, memory

    control, memory = jax.lax.fori_loop(
        0, STEPS, _step, (control, memory))
    out_ref[...] = memory


def kernel(context, question, knowledge, params):
    b = question.shape[0]
    c, r, w = params['ctrl'], params['read'], params['write']
    mm, km, rm, wm = (r['mem_mask'], r['know_mask'], r['concat_mask'],
                      w['concat_mask'])

    def row(v):
        return v.reshape(1, -1)

    prep = pl.pallas_call(
        _prep_kernel,
        out_shape=(
            jax.ShapeDtypeStruct((STEPS, b, DIM), F32),
            jax.ShapeDtypeStruct((HEADS, DH, DIM), F32),
            jax.ShapeDtypeStruct((HEADS, DH, DIM), F32),
            jax.ShapeDtypeStruct((HEADS, DH, DIM), F32),
            jax.ShapeDtypeStruct((HEADS, DH, DIM), F32),
        ),
        compiler_params=pltpu.CompilerParams(
            vmem_limit_bytes=60 * 1024 * 1024),
        name="mac_prep",
    )
    pa_all, khm, khk, khr, khw = prep(
        question, c['pos_w'], c['pos_b'].reshape(STEPS, 1, DIM),
        r['mem_w'], r['know_w'], r['concat_w'], r['cpm_w'], row(r['cpm_b']),
        w['concat_w'], w['cpm_w'], row(w['cpm_b']),
        mm['wk'].reshape(HEADS, DH, DIM), mm['bk'].reshape(HEADS, DH, 1),
        km['wk'].reshape(HEADS, DH, DIM), km['bk'].reshape(HEADS, DH, 1),
        rm['wk'].reshape(HEADS, DH, DIM), rm['bk'].reshape(HEADS, DH, 1),
        wm['wk'].reshape(HEADS, DH, DIM), wm['bk'].reshape(HEADS, DH, 1),
    )

    nb = b // BB
    S = context.shape[1]
    K = knowledge.shape[2]
    vmem = pl.BlockSpec(memory_space=pltpu.VMEM)
    out = pl.pallas_call(
        _mac_kernel,
        grid=(nb,),
        in_specs=[
            pl.BlockSpec((BB, DIM, S), lambda i: (i, 0, 0)),
            vmem,
            pl.BlockSpec(memory_space=pl.ANY),
        ] + [vmem] * 26,
        out_specs=pl.BlockSpec((BB, DIM), lambda i: (i, 0)),
        out_shape=jax.ShapeDtypeStruct((b, DIM), F32),
        scratch_shapes=[pltpu.VMEM((BB, DIM, K), BF16),
                        pltpu.SemaphoreType.DMA],
        compiler_params=pltpu.CompilerParams(
            dimension_semantics=("parallel",),
            vmem_limit_bytes=60 * 1024 * 1024),
        name="mac_steps",
    )
    return out(
        context.transpose(0, 2, 1), pa_all, knowledge.astype(BF16),
        params['control_0'], params['mem_0'],
        c['cq_w'][:, :DIM].T, c['cq_w'][:, DIM:].T, row(c['cq_b']),
        c['attn_w'],
        r['mem_w'].T, row(r['mem_b']),
        mm['wq'].T, row(mm['bq']), khm,
        km['wq'].T, row(km['bq']), khk,
        rm['wq'].T, row(rm['bq']), khr,
        wm['wq'].T, row(wm['bq']), khw,
        r['know_w'], r['concat_w'], r['attn_w'],
        w['concat_w'][:, :DIM].T, w['concat_w'][:, DIM:].T, row(w['concat_b']))
